# baseline (speedup 1.0000x reference)
import functools

import jax
import jax.numpy as jnp
from jax.experimental import pallas as pl
from jax.experimental.pallas import tpu as pltpu

D_MODEL = 1024
HIDDEN = 256
N_EXPERTS = 16
TOK_BLK = 512
NT = 4096 // TOK_BLK


def _ffn_body(x_ref, wsi_ref, wso_ref, w1_ref, w2_hbm, wg_ref, gb_ref, out_ref,
              wsib_ref, wsob_ref, w1b_ref, w2f_ref, w2b_ref,
              hbuf_ref, wmbuf_ref, accbuf_ref, sem):
    i = pl.program_id(0)

    @pl.when(i == 0)
    def _prologue():
        pltpu.make_async_copy(w2_hbm, w2f_ref, sem).start()
        wsib_ref[...] = wsi_ref[...].astype(jnp.bfloat16)
        wsob_ref[...] = wso_ref[...].astype(jnp.bfloat16)
        w1b_ref[...] = w1_ref[...].astype(jnp.bfloat16)

    @pl.when(i < NT)
    def _stage_a():
        xf = x_ref[...]
        xb = xf.astype(jnp.bfloat16)

        logits = jnp.dot(xf, wg_ref[...], preferred_element_type=jnp.float32)
        logits = logits + gb_ref[...][None, :]
        mx = jnp.max(logits, axis=-1, keepdims=True)
        el = jnp.exp(logits - mx)
        probs = el / jnp.sum(el, axis=-1, keepdims=True)
        p1 = jnp.max(probs, axis=-1, keepdims=True)
        masked = jnp.where(probs >= p1, -1.0, probs)
        p2 = jnp.max(masked, axis=-1, keepdims=True)
        wmat = jnp.where(probs >= p2, probs, 0.0)
        wmbuf_ref[i % 2] = wmat.astype(jnp.bfloat16)

        t = jnp.dot(xb, wsib_ref[...], preferred_element_type=jnp.float32)
        si = (jax.nn.silu(t[:, :HIDDEN]) * t[:, HIDDEN:]).astype(jnp.bfloat16)
        accbuf_ref[i % 2] = jnp.dot(si, wsob_ref[...],
                                    preferred_element_type=jnp.float32)

        t1 = jnp.dot(xb, w1b_ref[...], preferred_element_type=jnp.float32)
        hbuf_ref[i % 2] = (jax.nn.silu(t1[:, :HIDDEN])
                           * t1[:, HIDDEN:]).astype(jnp.bfloat16)

    @pl.when(i == 1)
    def _land_w2():
        pltpu.make_async_copy(w2_hbm, w2f_ref, sem).wait()
        for e in range(N_EXPERTS):
            w2b_ref[e * HIDDEN:(e + 1) * HIDDEN, :] = (
                w2f_ref[e].astype(jnp.bfloat16))

    @pl.when(i > 0)
    def _stage_b():
        j = (i - 1) % 2
        h = hbuf_ref[j]
        wm = wmbuf_ref[j]
        g = jnp.concatenate([h * wm[:, e:e + 1] for e in range(N_EXPERTS)],
                            axis=1)
        out_ref[...] = accbuf_ref[j] + jnp.dot(
            g, w2b_ref[...], preferred_element_type=jnp.float32)

    @pl.when(i == 0)
    def _dummy_out():
        out_ref[...] = jnp.zeros_like(out_ref)


@functools.partial(jax.jit, static_argnames=())
def _ffn(x_flat, wsi, wso, w1, w2, wg, gb):
    S = x_flat.shape[0]
    grid = (NT + 1,)
    return pl.pallas_call(
        _ffn_body,
        grid=grid,
        in_specs=[
            pl.BlockSpec((TOK_BLK, D_MODEL), lambda i: (jnp.minimum(i, NT - 1), 0)),
            pl.BlockSpec((D_MODEL, 2 * HIDDEN), lambda i: (0, 0)),
            pl.BlockSpec((HIDDEN, D_MODEL), lambda i: (0, 0)),
            pl.BlockSpec((D_MODEL, 2 * HIDDEN), lambda i: (0, 0)),
            pl.BlockSpec(memory_space=pl.ANY),
            pl.BlockSpec((D_MODEL, N_EXPERTS), lambda i: (0, 0)),
            pl.BlockSpec((N_EXPERTS,), lambda i: (0,)),
        ],
        out_specs=pl.BlockSpec((TOK_BLK, D_MODEL),
                               lambda i: (jnp.maximum(i - 1, 0), 0)),
        out_shape=jax.ShapeDtypeStruct((S, D_MODEL), jnp.float32),
        scratch_shapes=[
            pltpu.VMEM((D_MODEL, 2 * HIDDEN), jnp.bfloat16),
            pltpu.VMEM((HIDDEN, D_MODEL), jnp.bfloat16),
            pltpu.VMEM((D_MODEL, 2 * HIDDEN), jnp.bfloat16),
            pltpu.VMEM((N_EXPERTS, HIDDEN, D_MODEL), jnp.float32),
            pltpu.VMEM((N_EXPERTS * HIDDEN, D_MODEL), jnp.bfloat16),
            pltpu.VMEM((2, TOK_BLK, HIDDEN), jnp.bfloat16),
            pltpu.VMEM((2, TOK_BLK, N_EXPERTS), jnp.bfloat16),
            pltpu.VMEM((2, TOK_BLK, D_MODEL), jnp.float32),
            pltpu.SemaphoreType.DMA,
        ],
    )(x_flat, wsi, wso, w1, w2, wg, gb)


def kernel(x, W_shared_in, W_shared_out, W1, W2, Wg, gate_bias):
    B, T, C = x.shape
    flat = x.reshape(B * T, C)
    out = _ffn(flat, W_shared_in, W_shared_out, W1, W2, Wg, gate_bias)
    return out.reshape(B, T, C)


# speedup vs baseline: 1.0373x; 1.0373x over previous
import functools

import jax
import jax.numpy as jnp
from jax.experimental import pallas as pl
from jax.experimental.pallas import tpu as pltpu

D_MODEL = 1024
HIDDEN = 256
N_EXPERTS = 16
TOK_BLK = 512


def _ffn_body(x_ref, wsi_ref, wso_ref, w1_ref, w2_hbm, wg_ref, gb_ref, out_ref,
              wsib_ref, wsob_ref, w1b_ref, w2f_ref, w2b_ref, sem):
    i = pl.program_id(0)

    @pl.when(i == 0)
    def _prologue():
        pltpu.make_async_copy(w2_hbm, w2f_ref, sem).start()
        wsib_ref[...] = wsi_ref[...].astype(jnp.bfloat16)
        wsob_ref[...] = wso_ref[...].astype(jnp.bfloat16)
        w1b_ref[...] = w1_ref[...].astype(jnp.bfloat16)

    xf = x_ref[...]
    xb = xf.astype(jnp.bfloat16)

    logits = jnp.dot(xf, wg_ref[...], preferred_element_type=jnp.float32)
    logits = logits + gb_ref[...][None, :]
    mx = jnp.max(logits, axis=-1, keepdims=True)
    el = jnp.exp(logits - mx)
    probs = el / jnp.sum(el, axis=-1, keepdims=True)
    p1 = jnp.max(probs, axis=-1, keepdims=True)
    masked = jnp.where(probs >= p1, -1.0, probs)
    p2 = jnp.max(masked, axis=-1, keepdims=True)
    wmat = jnp.where(probs >= p2, probs, 0.0)
    wmat_b = wmat.astype(jnp.bfloat16)

    t = jnp.dot(xb, wsib_ref[...], preferred_element_type=jnp.float32)
    si = (jax.nn.silu(t[:, :HIDDEN]) * t[:, HIDDEN:]).astype(jnp.bfloat16)
    acc = jnp.dot(si, wsob_ref[...], preferred_element_type=jnp.float32)

    t1 = jnp.dot(xb, w1b_ref[...], preferred_element_type=jnp.float32)
    h = (jax.nn.silu(t1[:, :HIDDEN]) * t1[:, HIDDEN:]).astype(jnp.bfloat16)

    @pl.when(i == 0)
    def _land_w2():
        pltpu.make_async_copy(w2_hbm, w2f_ref, sem).wait()
        for e in range(N_EXPERTS):
            w2b_ref[e * HIDDEN:(e + 1) * HIDDEN, :] = (
                w2f_ref[e].astype(jnp.bfloat16))

    g = jnp.concatenate([h * wmat_b[:, e:e + 1] for e in range(N_EXPERTS)],
                        axis=1)
    acc = acc + jnp.dot(g, w2b_ref[...], preferred_element_type=jnp.float32)

    out_ref[...] = acc


@functools.partial(jax.jit, static_argnames=())
def _ffn(x_flat, wsi, wso, w1, w2, wg, gb):
    S = x_flat.shape[0]
    grid = (S // TOK_BLK,)
    return pl.pallas_call(
        _ffn_body,
        grid=grid,
        in_specs=[
            pl.BlockSpec((TOK_BLK, D_MODEL), lambda i: (i, 0)),
            pl.BlockSpec((D_MODEL, 2 * HIDDEN), lambda i: (0, 0)),
            pl.BlockSpec((HIDDEN, D_MODEL), lambda i: (0, 0)),
            pl.BlockSpec((D_MODEL, 2 * HIDDEN), lambda i: (0, 0)),
            pl.BlockSpec(memory_space=pl.ANY),
            pl.BlockSpec((D_MODEL, N_EXPERTS), lambda i: (0, 0)),
            pl.BlockSpec((N_EXPERTS,), lambda i: (0,)),
        ],
        out_specs=pl.BlockSpec((TOK_BLK, D_MODEL), lambda i: (i, 0)),
        out_shape=jax.ShapeDtypeStruct((S, D_MODEL), jnp.float32),
        scratch_shapes=[
            pltpu.VMEM((D_MODEL, 2 * HIDDEN), jnp.bfloat16),
            pltpu.VMEM((HIDDEN, D_MODEL), jnp.bfloat16),
            pltpu.VMEM((D_MODEL, 2 * HIDDEN), jnp.bfloat16),
            pltpu.VMEM((N_EXPERTS, HIDDEN, D_MODEL), jnp.float32),
            pltpu.VMEM((N_EXPERTS * HIDDEN, D_MODEL), jnp.bfloat16),
            pltpu.SemaphoreType.DMA,
        ],
    )(x_flat, wsi, wso, w1, w2, wg, gb)


def kernel(x, W_shared_in, W_shared_out, W1, W2, Wg, gate_bias):
    B, T, C = x.shape
    flat = x.reshape(B * T, C)
    out = _ffn(flat, W_shared_in, W_shared_out, W1, W2, Wg, gate_bias)
    return out.reshape(B, T, C)


# speedup vs baseline: 1.0526x; 1.0148x over previous
import functools

import jax
import jax.numpy as jnp
from jax.experimental import pallas as pl
from jax.experimental.pallas import tpu as pltpu

D_MODEL = 1024
HIDDEN = 256
N_EXPERTS = 16
TOK_BLK = 512


def _ffn_body(x_ref, wsi_ref, wso_ref, w1_ref, w2_ref, wg_ref, gb_ref, out_ref,
              wsib_ref, wsob_ref, w1b_ref, w2b_ref):
    i = pl.program_id(0)

    @pl.when(i == 0)
    def _prologue():
        wsib_ref[...] = wsi_ref[...].astype(jnp.bfloat16)
        wsob_ref[...] = wso_ref[...].astype(jnp.bfloat16)
        w1b_ref[...] = w1_ref[...].astype(jnp.bfloat16)
        for e in range(N_EXPERTS):
            w2b_ref[e * HIDDEN:(e + 1) * HIDDEN, :] = (
                w2_ref[e].astype(jnp.bfloat16))

    xf = x_ref[...]
    xb = xf.astype(jnp.bfloat16)

    logits = jnp.dot(xf, wg_ref[...], preferred_element_type=jnp.float32)
    logits = logits + gb_ref[...][None, :]
    mx = jnp.max(logits, axis=-1, keepdims=True)
    el = jnp.exp(logits - mx)
    probs = el / jnp.sum(el, axis=-1, keepdims=True)
    p1 = jnp.max(probs, axis=-1, keepdims=True)
    masked = jnp.where(probs >= p1, -1.0, probs)
    p2 = jnp.max(masked, axis=-1, keepdims=True)
    wmat = jnp.where(probs >= p2, probs, 0.0)
    wmat_b = wmat.astype(jnp.bfloat16)

    t = jnp.dot(xb, wsib_ref[...], preferred_element_type=jnp.float32)
    si = (jax.nn.silu(t[:, :HIDDEN]) * t[:, HIDDEN:]).astype(jnp.bfloat16)
    acc = jnp.dot(si, wsob_ref[...], preferred_element_type=jnp.float32)

    t1 = jnp.dot(xb, w1b_ref[...], preferred_element_type=jnp.float32)
    h = (jax.nn.silu(t1[:, :HIDDEN]) * t1[:, HIDDEN:]).astype(jnp.bfloat16)

    g_lo = jnp.concatenate([h * wmat_b[:, e:e + 1] for e in range(8)],
                           axis=1)
    acc = acc + jnp.dot(g_lo, w2b_ref[:8 * HIDDEN, :],
                        preferred_element_type=jnp.float32)
    g_hi = jnp.concatenate([h * wmat_b[:, e:e + 1] for e in range(8, 16)],
                           axis=1)
    acc = acc + jnp.dot(g_hi, w2b_ref[8 * HIDDEN:, :],
                        preferred_element_type=jnp.float32)

    out_ref[...] = acc


@functools.partial(jax.jit, static_argnames=())
def _ffn(x_flat, wsi, wso, w1, w2, wg, gb):
    S = x_flat.shape[0]
    grid = (S // TOK_BLK,)
    return pl.pallas_call(
        _ffn_body,
        grid=grid,
        in_specs=[
            pl.BlockSpec((TOK_BLK, D_MODEL), lambda i: (i, 0)),
            pl.BlockSpec((D_MODEL, 2 * HIDDEN), lambda i: (0, 0)),
            pl.BlockSpec((HIDDEN, D_MODEL), lambda i: (0, 0)),
            pl.BlockSpec((D_MODEL, 2 * HIDDEN), lambda i: (0, 0)),
            pl.BlockSpec((N_EXPERTS, HIDDEN, D_MODEL), lambda i: (0, 0, 0)),
            pl.BlockSpec((D_MODEL, N_EXPERTS), lambda i: (0, 0)),
            pl.BlockSpec((N_EXPERTS,), lambda i: (0,)),
        ],
        out_specs=pl.BlockSpec((TOK_BLK, D_MODEL), lambda i: (i, 0)),
        out_shape=jax.ShapeDtypeStruct((S, D_MODEL), jnp.float32),
        scratch_shapes=[
            pltpu.VMEM((D_MODEL, 2 * HIDDEN), jnp.bfloat16),
            pltpu.VMEM((HIDDEN, D_MODEL), jnp.bfloat16),
            pltpu.VMEM((D_MODEL, 2 * HIDDEN), jnp.bfloat16),
            pltpu.VMEM((N_EXPERTS * HIDDEN, D_MODEL), jnp.bfloat16),
        ],
    )(x_flat, wsi, wso, w1, w2, wg, gb)


def kernel(x, W_shared_in, W_shared_out, W1, W2, Wg, gate_bias):
    B, T, C = x.shape
    flat = x.reshape(B * T, C)
    out = _ffn(flat, W_shared_in, W_shared_out, W1, W2, Wg, gate_bias)
    return out.reshape(B, T, C)
